# baseline (speedup 1.0000x reference)
import functools
import math

import jax
import jax.numpy as jnp
from jax import lax
from jax.experimental import pallas as pl
from jax.experimental.pallas import tpu as pltpu
from jax.experimental.pallas import tpu_sc as plsc

_B = 4
_S = 4096
_D = 1024
_ROWS = _B * _S
_NW = 32
_ROWS_PER_W = _ROWS // _NW
_CHUNK = 32
_N_ROUNDS = _ROWS_PER_W // _CHUNK
_VECS_PER_ROW = _D // 16
_SCALE = math.sqrt(_D)

_mesh = plsc.VectorSubcoreMesh(core_axis_name="c", subcore_axis_name="s")


@functools.partial(
    pl.kernel,
    out_type=jax.ShapeDtypeStruct((_ROWS, _D), jnp.float32),
    mesh=_mesh,
    scratch_types=[
        pltpu.VMEM((_CHUNK,), jnp.int32),
        pltpu.VMEM((_CHUNK, _D), jnp.float32),
        pltpu.VMEM((_CHUNK, _D), jnp.float32),
        pltpu.SemaphoreType.DMA,
    ],
)
def _embed(tokens_hbm, table_hbm, pos_hbm, out_hbm, idx_v, tok_v, pos_v, sem):
    wid = lax.axis_index("s") * 2 + lax.axis_index("c")
    base = wid * _ROWS_PER_W
    pos_base = base % _S

    for r in range(_N_ROUNDS):
        off = base + r * _CHUNK
        pltpu.sync_copy(tokens_hbm.at[pl.ds(off, _CHUNK)], idx_v)
        gather = pltpu.async_copy(table_hbm.at[idx_v], tok_v, sem)
        pltpu.sync_copy(pos_hbm.at[pl.ds(pos_base + r * _CHUNK, _CHUNK)], pos_v)
        gather.wait()

        def row_body(i, _):
            def vec_body(j, _):
                col = j * 16
                tok = tok_v[i, pl.ds(col, 16)]
                ps = pos_v[i, pl.ds(col, 16)]
                tok_v[i, pl.ds(col, 16)] = (tok + ps) * _SCALE
                return 0

            return lax.fori_loop(0, _VECS_PER_ROW, vec_body, 0)

        lax.fori_loop(0, _CHUNK, row_body, 0)
        pltpu.sync_copy(tok_v, out_hbm.at[pl.ds(off, _CHUNK)])


def kernel(tokens, token_table, pos_table):
    batch, seq = tokens.shape
    flat = tokens.reshape(-1).astype(jnp.int32)
    out = _embed(flat, token_table, pos_table)
    return out.reshape(batch, seq, _D)


# speedup vs baseline: 3.0393x; 3.0393x over previous
import functools
import math

import jax
import jax.numpy as jnp
from jax import lax
from jax.experimental import pallas as pl
from jax.experimental.pallas import tpu as pltpu
from jax.experimental.pallas import tpu_sc as plsc

_B = 4
_S = 4096
_D = 1024
_ROWS = _B * _S
_NW = 32
_ROWS_PER_W = _ROWS // _NW
_CHUNK = 16
_N_ROUNDS = _ROWS_PER_W // _CHUNK
_VECS_PER_ROW = _D // 16
_SCALE = math.sqrt(_D)

_mesh = plsc.VectorSubcoreMesh(core_axis_name="c", subcore_axis_name="s")


@functools.partial(
    pl.kernel,
    out_type=jax.ShapeDtypeStruct((_ROWS, _D), jnp.float32),
    mesh=_mesh,
    scratch_types=[
        pltpu.VMEM((_ROWS_PER_W,), jnp.int32),
        pltpu.VMEM((_CHUNK, _D), jnp.float32),
        pltpu.VMEM((_CHUNK, _D), jnp.float32),
        pltpu.VMEM((_CHUNK, _D), jnp.float32),
        pltpu.VMEM((_CHUNK, _D), jnp.float32),
        pltpu.VMEM((_CHUNK, _D), jnp.float32),
        pltpu.VMEM((_CHUNK, _D), jnp.float32),
        pltpu.SemaphoreType.DMA,
        pltpu.SemaphoreType.DMA,
        pltpu.SemaphoreType.DMA,
        pltpu.SemaphoreType.DMA,
        pltpu.SemaphoreType.DMA,
        pltpu.SemaphoreType.DMA,
    ],
)
def _embed(tokens_hbm, table_hbm, pos_hbm, out_hbm, idx_all,
           tok0, tok1, pos0, pos1, stg0, stg1,
           gsem0, gsem1, psem0, psem1, osem0, osem1):
    wid = lax.axis_index("s") * 2 + lax.axis_index("c")
    base = wid * _ROWS_PER_W
    pos_base = base % _S

    bufs = (
        (tok0, pos0, stg0, gsem0, psem0, osem0),
        (tok1, pos1, stg1, gsem1, psem1, osem1),
    )

    pltpu.sync_copy(tokens_hbm.at[pl.ds(base, _ROWS_PER_W)], idx_all)

    def issue(r, tok, pos, gsem, psem):
        idx_sl = idx_all.at[pl.ds(r * _CHUNK, _CHUNK)]
        pltpu.async_copy(table_hbm.at[idx_sl], tok, gsem)
        pltpu.async_copy(pos_hbm.at[pl.ds(pos_base + r * _CHUNK, _CHUNK)],
                         pos, psem)

    for j, (tok, pos, stg, gsem, psem, osem) in enumerate(bufs):
        issue(j, tok, pos, gsem, psem)

    def loop_body(g, _):
        for j, (tok, pos, stg, gsem, psem, osem) in enumerate(bufs):
            r = 2 * g + j
            idx_sl = idx_all.at[pl.ds(r * _CHUNK, _CHUNK)]
            pltpu.make_async_copy(table_hbm.at[idx_sl], tok, gsem).wait()
            pltpu.make_async_copy(
                pos_hbm.at[pl.ds(0, _CHUNK)], pos, psem).wait()

            @pl.when(g > 0)
            def _():
                pltpu.make_async_copy(
                    stg, out_hbm.at[pl.ds(0, _CHUNK)], osem).wait()

            def row_body(i, _):
                for v in range(_VECS_PER_ROW):
                    col = v * 16
                    stg[i, pl.ds(col, 16)] = (
                        tok[i, pl.ds(col, 16)] + pos[i, pl.ds(col, 16)]
                    ) * _SCALE
                return 0

            lax.fori_loop(0, _CHUNK, row_body, 0)

            @pl.when(r + 2 < _N_ROUNDS)
            def _():
                issue(r + 2, tok, pos, gsem, psem)

            pltpu.async_copy(stg, out_hbm.at[pl.ds(base + r * _CHUNK, _CHUNK)],
                             osem)
        return 0

    lax.fori_loop(0, _N_ROUNDS // 2, loop_body, 0)

    for j, (tok, pos, stg, gsem, psem, osem) in enumerate(bufs):
        pltpu.make_async_copy(stg, out_hbm.at[pl.ds(0, _CHUNK)], osem).wait()


def kernel(tokens, token_table, pos_table):
    batch, seq = tokens.shape
    flat = tokens.reshape(-1).astype(jnp.int32)
    out = _embed(flat, token_table, pos_table)
    return out.reshape(batch, seq, _D)
